# Initial kernel scaffold; baseline (speedup 1.0000x reference)
import jax
import jax.numpy as jnp
from jax.experimental import pallas as pl


def kernel(rm, psm, pos_equal_one, neg_equal_one, targets):
    raise NotImplementedError("write your pallas kernel here")


# speedup vs baseline: 11.0873x; 11.0873x over previous
import jax
import jax.numpy as jnp
from jax.experimental import pallas as pl
from jax.experimental.pallas import tpu as pltpu

_NEG_RATIO = 1.0
_ALPHA = 1.5
_BETA = 1.0
_GAMMA = 2.0

_B, _H, _W, _A = 8, 200, 176, 10
_C = _A * 7
_N = _B * _H * _W * _A
_LANES = 128
_ROWS = _N // _LANES
_CHUNK = 200
_GRID_A = _ROWS // _CHUNK
_HB = 40
_GH = _H // _HB

_INT_MIN = -2147483648
_POS_MASK = 0x7FFFFFFF


def _monokey(x):
    b = jax.lax.bitcast_convert_type(x, jnp.int32)
    return jnp.where(b < 0, b ^ _POS_MASK, b)


def _unkey(k):
    b = jnp.where(k < 0, k ^ _POS_MASK, k)
    return jax.lax.bitcast_convert_type(b, jnp.float32)


def _cls_sel_kernel(psm_ref, pos_ref, neg_ref, out_ref, key_ref, acc_ref):
    i = pl.program_id(0)

    @pl.when(i == 0)
    def _init():
        acc_ref[0] = 0.0
        acc_ref[1] = 0.0
        out_ref[...] = jnp.zeros_like(out_ref)

    x = psm_ref[...]
    posb = pos_ref[...]
    negb = neg_ref[...]
    p = jax.nn.sigmoid(x)
    acc_ref[0] += jnp.sum(posb)
    acc_ref[1] += jnp.sum(-posb * jnp.log(p + 1e-6))
    v = -negb * jnp.log(1.0 - p + 1e-6)
    key_ref[pl.ds(i * _CHUNK, _CHUNK), :] = _monokey(v)

    @pl.when(i == _GRID_A - 1)
    def _finish():
        pos_sum = acc_ref[0]
        k_i = jnp.floor(_NEG_RATIO * (pos_sum + 1.0)).astype(jnp.int32)
        k_eff = jnp.minimum(k_i, _N)

        def count_ge(thresh):
            def body(c, acc):
                ik = key_ref[pl.ds(c * _CHUNK, _CHUNK), :]
                return acc + jnp.sum((ik >= thresh).astype(jnp.int32))
            return jax.lax.fori_loop(0, _GRID_A, body, jnp.int32(0))

        def bit_step(b, upfx):
            m = jax.lax.shift_left(jnp.int32(1), jnp.int32(31) - b)
            ucand = upfx | m
            cand = ucand ^ _INT_MIN
            cnt = count_ge(cand)
            return jnp.where(cnt >= k_eff, ucand, upfx)

        upfx = jax.lax.fori_loop(0, 32, bit_step, jnp.int32(0))
        thresh = upfx ^ _INT_MIN

        def tail(c, carry):
            cg, sg = carry
            ik = key_ref[pl.ds(c * _CHUNK, _CHUNK), :]
            gt = ik > thresh
            vv = _unkey(ik)
            return (cg + jnp.sum(gt.astype(jnp.int32)),
                    sg + jnp.sum(jnp.where(gt, vv, 0.0)))

        cnt_gt, sum_gt = jax.lax.fori_loop(
            0, _GRID_A, tail, (jnp.int32(0), jnp.float32(0.0)))

        t_val = _unkey(thresh)
        topk_sum = sum_gt + (k_eff - cnt_gt).astype(jnp.float32) * t_val
        denom = k_eff.astype(jnp.float32)

        lane = jax.lax.broadcasted_iota(jnp.int32, (1, _LANES), 1)
        row = jnp.where(lane == 0, pos_sum,
              jnp.where(lane == 1, acc_ref[1],
              jnp.where(lane == 2, topk_sum,
              jnp.where(lane == 3, denom, 0.0))))
        out_ref[...] = row


def _reg_kernel(rm_ref, tg_ref, pos_ref, out_ref, acc_ref):
    b = pl.program_id(0)
    a = pl.program_id(1)
    h = pl.program_id(2)

    @pl.when((b == 0) & (a == 0) & (h == 0))
    def _init():
        acc_ref[0] = 0.0
        out_ref[...] = jnp.zeros_like(out_ref)

    x = rm_ref[0]
    y = tg_ref[0]
    pm = pos_ref[0, 0]
    d = (x - y) * pm[None, :, :]
    ad = jnp.abs(d)
    f = jnp.where(ad < 1.0, 0.5 * d * d, ad - 0.5)
    acc_ref[0] += jnp.sum(f)

    @pl.when((b == _B - 1) & (a == _A - 1) & (h == _GH - 1))
    def _fin():
        out_ref[...] = jnp.full_like(out_ref, acc_ref[0])


def kernel(rm, psm, pos_equal_one, neg_equal_one, targets):
    psm_t = jnp.transpose(psm, (0, 2, 3, 1)).reshape(_ROWS, _LANES)
    pos_r = pos_equal_one.reshape(_ROWS, _LANES)
    neg_r = neg_equal_one.reshape(_ROWS, _LANES)

    row = pl.pallas_call(
        _cls_sel_kernel,
        grid=(_GRID_A,),
        in_specs=[pl.BlockSpec((_CHUNK, _LANES), lambda i: (i, 0))] * 3,
        out_specs=pl.BlockSpec((1, _LANES), lambda i: (0, 0)),
        out_shape=jax.ShapeDtypeStruct((1, _LANES), jnp.float32),
        scratch_shapes=[pltpu.VMEM((_ROWS, _LANES), jnp.int32),
                        pltpu.SMEM((2,), jnp.float32)],
    )(psm_t, pos_r, neg_r)

    tg_t = jnp.transpose(targets, (0, 3, 1, 2))
    pos_t = jnp.transpose(pos_equal_one, (0, 3, 1, 2))

    reg = pl.pallas_call(
        _reg_kernel,
        grid=(_B, _A, _GH),
        in_specs=[
            pl.BlockSpec((1, 7, _HB, _W), lambda b, a, h: (b, a, h, 0)),
            pl.BlockSpec((1, 7, _HB, _W), lambda b, a, h: (b, a, h, 0)),
            pl.BlockSpec((1, 1, _HB, _W), lambda b, a, h: (b, a, h, 0)),
        ],
        out_specs=pl.BlockSpec((1, 1), lambda b, a, h: (0, 0)),
        out_shape=jax.ShapeDtypeStruct((1, 1), jnp.float32),
        scratch_shapes=[pltpu.SMEM((1,), jnp.float32)],
    )(rm, tg_t, pos_t)

    pos_sum = row[0, 0]
    clsp_sum = row[0, 1]
    topk_sum = row[0, 2]
    denom = row[0, 3]
    reg_sum = reg[0, 0]

    cls_pos_loss = _ALPHA * (clsp_sum / (pos_sum + 1e-6))
    cls_neg_loss = _BETA * (topk_sum / (denom + 1e-6))
    reg_loss = _GAMMA * (reg_sum / (pos_sum + 1e-6))
    conf_loss = cls_pos_loss + cls_neg_loss
    return (conf_loss, reg_loss, cls_pos_loss, cls_neg_loss)


# speedup vs baseline: 12.9896x; 1.1716x over previous
import jax
import jax.numpy as jnp
from jax.experimental import pallas as pl
from jax.experimental.pallas import tpu as pltpu

_NEG_RATIO = 1.0
_ALPHA = 1.5
_BETA = 1.0
_GAMMA = 2.0

_B, _H, _W, _A = 8, 200, 176, 10
_C = _A * 7
_N = _B * _H * _W * _A
_KROWS = _B * _A * _H
_BH2 = 40
_GH2 = _H // _BH2
_CH = 200
_NCH = _KROWS // _CH
_HB = 8
_GH = _H // _HB

_INT_MIN = -2147483648
_POS_MASK = 0x7FFFFFFF


def _monokey(x):
    b = jax.lax.bitcast_convert_type(x, jnp.int32)
    return jnp.where(b < 0, b ^ _POS_MASK, b)


def _unkey(k):
    b = jnp.where(k < 0, k ^ _POS_MASK, k)
    return jax.lax.bitcast_convert_type(b, jnp.float32)


def _cls_sel_kernel(psm_ref, pos_ref, neg_ref, out_ref, key_ref, acc_ref):
    b = pl.program_id(0)
    h = pl.program_id(1)

    @pl.when((b == 0) & (h == 0))
    def _init():
        acc_ref[0] = 0.0
        acc_ref[1] = 0.0
        out_ref[...] = jnp.zeros_like(out_ref)

    x = psm_ref[0]
    post = jnp.transpose(pos_ref[0], (2, 0, 1))
    negt = jnp.transpose(neg_ref[0], (2, 0, 1))
    p = jax.nn.sigmoid(x)
    acc_ref[0] += jnp.sum(post)
    acc_ref[1] += jnp.sum(-post * jnp.log(p + 1e-6))
    v = -negt * jnp.log(1.0 - p + 1e-6)
    keys = _monokey(v)
    base = b * (_A * _H) + h * _BH2
    for c in range(_A):
        key_ref[pl.ds(base + c * _H, _BH2), :] = keys[c]

    @pl.when((b == _B - 1) & (h == _GH2 - 1))
    def _finish():
        pos_sum = acc_ref[0]
        k_i = jnp.floor(_NEG_RATIO * (pos_sum + 1.0)).astype(jnp.int32)
        k_eff = jnp.minimum(k_i, _N)

        def count_ge(thresh):
            def body(c, acc):
                ik = key_ref[pl.ds(c * _CH, _CH), :]
                return acc + jnp.sum((ik >= thresh).astype(jnp.int32))
            return jax.lax.fori_loop(0, _NCH, body, jnp.int32(0))

        def bit_step(bit, upfx):
            m = jax.lax.shift_left(jnp.int32(1), jnp.int32(31) - bit)
            ucand = upfx | m
            cand = ucand ^ _INT_MIN
            cnt = count_ge(cand)
            return jnp.where(cnt >= k_eff, ucand, upfx)

        upfx = jax.lax.fori_loop(0, 32, bit_step, jnp.int32(0))
        thresh = upfx ^ _INT_MIN

        def tail(c, carry):
            cg, sg = carry
            ik = key_ref[pl.ds(c * _CH, _CH), :]
            gt = ik > thresh
            vv = _unkey(ik)
            return (cg + jnp.sum(gt.astype(jnp.int32)),
                    sg + jnp.sum(jnp.where(gt, vv, 0.0)))

        cnt_gt, sum_gt = jax.lax.fori_loop(
            0, _NCH, tail, (jnp.int32(0), jnp.float32(0.0)))

        t_val = _unkey(thresh)
        topk_sum = sum_gt + (k_eff - cnt_gt).astype(jnp.float32) * t_val
        denom = k_eff.astype(jnp.float32)

        lane = jax.lax.broadcasted_iota(jnp.int32, (1, 128), 1)
        row = jnp.where(lane == 0, pos_sum,
              jnp.where(lane == 1, acc_ref[1],
              jnp.where(lane == 2, topk_sum,
              jnp.where(lane == 3, denom, 0.0))))
        out_ref[...] = row


def _reg_kernel(rm_ref, tg_ref, pos_ref, out_ref, acc_ref):
    b = pl.program_id(0)
    h = pl.program_id(1)

    @pl.when((b == 0) & (h == 0))
    def _init():
        acc_ref[0] = 0.0
        out_ref[...] = jnp.zeros_like(out_ref)

    x = rm_ref[0]
    y = jnp.transpose(tg_ref[0], (2, 0, 1))
    pmt = jnp.transpose(pos_ref[0], (2, 0, 1))
    pm70 = jnp.broadcast_to(pmt[:, None], (_A, 7, _HB, _W)).reshape(_C, _HB, _W)
    d = (x - y) * pm70
    ad = jnp.abs(d)
    f = jnp.where(ad < 1.0, 0.5 * d * d, ad - 0.5)
    acc_ref[0] += jnp.sum(f)

    @pl.when((b == _B - 1) & (h == _GH - 1))
    def _fin():
        out_ref[...] = jnp.full_like(out_ref, acc_ref[0])


def kernel(rm, psm, pos_equal_one, neg_equal_one, targets):
    row = pl.pallas_call(
        _cls_sel_kernel,
        grid=(_B, _GH2),
        in_specs=[
            pl.BlockSpec((1, _A, _BH2, _W), lambda b, h: (b, 0, h, 0)),
            pl.BlockSpec((1, _BH2, _W, _A), lambda b, h: (b, h, 0, 0)),
            pl.BlockSpec((1, _BH2, _W, _A), lambda b, h: (b, h, 0, 0)),
        ],
        out_specs=pl.BlockSpec((1, 128), lambda b, h: (0, 0)),
        out_shape=jax.ShapeDtypeStruct((1, 128), jnp.float32),
        scratch_shapes=[pltpu.VMEM((_KROWS, _W), jnp.int32),
                        pltpu.SMEM((2,), jnp.float32)],
    )(psm, pos_equal_one, neg_equal_one)

    reg = pl.pallas_call(
        _reg_kernel,
        grid=(_B, _GH),
        in_specs=[
            pl.BlockSpec((1, _C, _HB, _W), lambda b, h: (b, 0, h, 0)),
            pl.BlockSpec((1, _HB, _W, _C), lambda b, h: (b, h, 0, 0)),
            pl.BlockSpec((1, _HB, _W, _A), lambda b, h: (b, h, 0, 0)),
        ],
        out_specs=pl.BlockSpec((1, 1), lambda b, h: (0, 0)),
        out_shape=jax.ShapeDtypeStruct((1, 1), jnp.float32),
        scratch_shapes=[pltpu.SMEM((1,), jnp.float32)],
    )(rm, targets, pos_equal_one)

    pos_sum = row[0, 0]
    clsp_sum = row[0, 1]
    topk_sum = row[0, 2]
    denom = row[0, 3]
    reg_sum = reg[0, 0]

    cls_pos_loss = _ALPHA * (clsp_sum / (pos_sum + 1e-6))
    cls_neg_loss = _BETA * (topk_sum / (denom + 1e-6))
    reg_loss = _GAMMA * (reg_sum / (pos_sum + 1e-6))
    conf_loss = cls_pos_loss + cls_neg_loss
    return (conf_loss, reg_loss, cls_pos_loss, cls_neg_loss)


# speedup vs baseline: 18.3084x; 1.4095x over previous
import jax
import jax.numpy as jnp
from jax.experimental import pallas as pl
from jax.experimental.pallas import tpu as pltpu

_NEG_RATIO = 1.0
_ALPHA = 1.5
_BETA = 1.0
_GAMMA = 2.0

_B, _H, _W, _A = 8, 200, 176, 10
_C = _A * 7
_N = _B * _H * _W * _A
_KROWS = _B * _A * _H
_BH2 = 40
_GH2 = _H // _BH2
_CH = 800
_NCH = _KROWS // _CH
_HB = 8
_GH = _H // _HB

_INT_MIN = -2147483648
_POS_MASK = 0x7FFFFFFF


def _monokey(x):
    b = jax.lax.bitcast_convert_type(x, jnp.int32)
    return jnp.where(b < 0, b ^ _POS_MASK, b)


def _unkey(k):
    b = jnp.where(k < 0, k ^ _POS_MASK, k)
    return jax.lax.bitcast_convert_type(b, jnp.float32)


def _cls_sel_kernel(psm_ref, pos_ref, neg_ref, out_ref, post_ref, key_ref,
                    acc_ref):
    b = pl.program_id(0)
    h = pl.program_id(1)

    @pl.when((b == 0) & (h == 0))
    def _init():
        acc_ref[0] = 0.0
        acc_ref[1] = 0.0
        out_ref[...] = jnp.zeros_like(out_ref)

    x = psm_ref[0]
    post = jnp.transpose(pos_ref[0], (2, 0, 1))
    negt = jnp.transpose(neg_ref[0], (2, 0, 1))
    post_ref[0] = post
    p = jax.nn.sigmoid(x)
    acc_ref[0] += jnp.sum(post)
    acc_ref[1] += jnp.sum(-post * jnp.log(p + 1e-6))
    v = -negt * jnp.log(1.0 - p + 1e-6)
    keys = _monokey(v)
    base = b * (_A * _H) + h * _BH2
    for c in range(_A):
        key_ref[pl.ds(base + c * _H, _BH2), :] = keys[c]

    @pl.when((b == _B - 1) & (h == _GH2 - 1))
    def _finish():
        pos_sum = acc_ref[0]
        k_i = jnp.floor(_NEG_RATIO * (pos_sum + 1.0)).astype(jnp.int32)
        k_eff = jnp.minimum(k_i, _N)

        def count_ge(thresh):
            def body(c, acc):
                ik = key_ref[pl.ds(c * _CH, _CH), :]
                return acc + jnp.sum((ik >= thresh).astype(jnp.int32))
            return jax.lax.fori_loop(0, _NCH, body, jnp.int32(0))

        def bit_cond(st):
            bit, _, cnt = st
            return (bit < 32) & (cnt != k_eff)

        def bit_step(st):
            bit, upfx, cnt = st
            m = jax.lax.shift_left(jnp.int32(1), jnp.int32(31) - bit)
            ucand = upfx | m
            cand = ucand ^ _INT_MIN
            c2 = count_ge(cand)
            take = c2 >= k_eff
            return (bit + jnp.int32(1),
                    jnp.where(take, ucand, upfx),
                    jnp.where(take, c2, cnt))

        _, upfx, _ = jax.lax.while_loop(
            bit_cond, bit_step, (jnp.int32(0), jnp.int32(0), jnp.int32(_N)))
        thresh = upfx ^ _INT_MIN

        def tail(c, carry):
            cg, sg = carry
            ik = key_ref[pl.ds(c * _CH, _CH), :]
            gt = ik > thresh
            vv = _unkey(ik)
            return (cg + jnp.sum(gt.astype(jnp.int32)),
                    sg + jnp.sum(jnp.where(gt, vv, 0.0)))

        cnt_gt, sum_gt = jax.lax.fori_loop(
            0, _NCH, tail, (jnp.int32(0), jnp.float32(0.0)))

        t_val = _unkey(thresh)
        topk_sum = sum_gt + (k_eff - cnt_gt).astype(jnp.float32) * t_val
        denom = k_eff.astype(jnp.float32)

        lane = jax.lax.broadcasted_iota(jnp.int32, (1, 128), 1)
        row = jnp.where(lane == 0, pos_sum,
              jnp.where(lane == 1, acc_ref[1],
              jnp.where(lane == 2, topk_sum,
              jnp.where(lane == 3, denom, 0.0))))
        out_ref[...] = row


def _reg_kernel(rm_ref, tg_ref, pos_ref, out_ref, acc_ref):
    b = pl.program_id(0)
    h = pl.program_id(1)

    @pl.when((b == 0) & (h == 0))
    def _init():
        acc_ref[0] = 0.0
        out_ref[...] = jnp.zeros_like(out_ref)

    x = rm_ref[0]
    y = jnp.transpose(tg_ref[0], (2, 0, 1))
    pmt = pos_ref[0]
    pm70 = jnp.broadcast_to(pmt[:, None], (_A, 7, _HB, _W)).reshape(_C, _HB, _W)
    d = (x - y) * pm70
    ad = jnp.abs(d)
    f = jnp.where(ad < 1.0, 0.5 * d * d, ad - 0.5)
    acc_ref[0] += jnp.sum(f)

    @pl.when((b == _B - 1) & (h == _GH - 1))
    def _fin():
        out_ref[...] = jnp.full_like(out_ref, acc_ref[0])


def kernel(rm, psm, pos_equal_one, neg_equal_one, targets):
    row = pl.pallas_call(
        _cls_sel_kernel,
        grid=(_B, _GH2),
        in_specs=[
            pl.BlockSpec((1, _A, _BH2, _W), lambda b, h: (b, 0, h, 0)),
            pl.BlockSpec((1, _BH2, _W, _A), lambda b, h: (b, h, 0, 0)),
            pl.BlockSpec((1, _BH2, _W, _A), lambda b, h: (b, h, 0, 0)),
        ],
        out_specs=[
            pl.BlockSpec((1, 128), lambda b, h: (0, 0)),
            pl.BlockSpec((1, _A, _BH2, _W), lambda b, h: (b, 0, h, 0)),
        ],
        out_shape=[
            jax.ShapeDtypeStruct((1, 128), jnp.float32),
            jax.ShapeDtypeStruct((_B, _A, _H, _W), jnp.float32),
        ],
        scratch_shapes=[pltpu.VMEM((_KROWS, _W), jnp.int32),
                        pltpu.SMEM((2,), jnp.float32)],
    )(psm, pos_equal_one, neg_equal_one)
    row, pos_t = row

    reg = pl.pallas_call(
        _reg_kernel,
        grid=(_B, _GH),
        in_specs=[
            pl.BlockSpec((1, _C, _HB, _W), lambda b, h: (b, 0, h, 0)),
            pl.BlockSpec((1, _HB, _W, _C), lambda b, h: (b, h, 0, 0)),
            pl.BlockSpec((1, _A, _HB, _W), lambda b, h: (b, 0, h, 0)),
        ],
        out_specs=pl.BlockSpec((1, 1), lambda b, h: (0, 0)),
        out_shape=jax.ShapeDtypeStruct((1, 1), jnp.float32),
        scratch_shapes=[pltpu.SMEM((1,), jnp.float32)],
    )(rm, targets, pos_t)

    pos_sum = row[0, 0]
    clsp_sum = row[0, 1]
    topk_sum = row[0, 2]
    denom = row[0, 3]
    reg_sum = reg[0, 0]

    cls_pos_loss = _ALPHA * (clsp_sum / (pos_sum + 1e-6))
    cls_neg_loss = _BETA * (topk_sum / (denom + 1e-6))
    reg_loss = _GAMMA * (reg_sum / (pos_sum + 1e-6))
    conf_loss = cls_pos_loss + cls_neg_loss
    return (conf_loss, reg_loss, cls_pos_loss, cls_neg_loss)


# speedup vs baseline: 20.4687x; 1.1180x over previous
import jax
import jax.numpy as jnp
from jax.experimental import pallas as pl
from jax.experimental.pallas import tpu as pltpu

_NEG_RATIO = 1.0
_ALPHA = 1.5
_BETA = 1.0
_GAMMA = 2.0

_B, _H, _W, _A = 8, 200, 176, 10
_C = _A * 7
_N = _B * _H * _W * _A
_KROWS = _B * _A * _H
_BH2 = 40
_GH2 = _H // _BH2
_CH = 800
_NCH = _KROWS // _CH
_HB = 40
_GH = _H // _HB

_INT_MIN = -2147483648
_POS_MASK = 0x7FFFFFFF


def _monokey(x):
    b = jax.lax.bitcast_convert_type(x, jnp.int32)
    return jnp.where(b < 0, b ^ _POS_MASK, b)


def _unkey(k):
    b = jnp.where(k < 0, k ^ _POS_MASK, k)
    return jax.lax.bitcast_convert_type(b, jnp.float32)


def _cls_sel_kernel(psm_ref, pos_ref, neg_ref, out_ref, post_ref, key_ref,
                    acc_ref):
    b = pl.program_id(0)
    h = pl.program_id(1)

    @pl.when((b == 0) & (h == 0))
    def _init():
        acc_ref[0] = 0.0
        acc_ref[1] = 0.0
        out_ref[...] = jnp.zeros_like(out_ref)

    x = psm_ref[0]
    post = jnp.transpose(pos_ref[0], (2, 0, 1))
    negt = jnp.transpose(neg_ref[0], (2, 0, 1))
    post_ref[0] = post
    p = jax.nn.sigmoid(x)
    acc_ref[0] += jnp.sum(post)
    acc_ref[1] += jnp.sum(-post * jnp.log(p + 1e-6))
    v = -negt * jnp.log(1.0 - p + 1e-6)
    keys = _monokey(v)
    base = b * (_A * _H) + h * _BH2
    for c in range(_A):
        key_ref[pl.ds(base + c * _H, _BH2), :] = keys[c]

    @pl.when((b == _B - 1) & (h == _GH2 - 1))
    def _finish():
        pos_sum = acc_ref[0]
        k_i = jnp.floor(_NEG_RATIO * (pos_sum + 1.0)).astype(jnp.int32)
        k_eff = jnp.minimum(k_i, _N)

        def count_ge(thresh):
            def body(c, acc):
                ik = key_ref[pl.ds(c * _CH, _CH), :]
                return acc + jnp.sum((ik >= thresh).astype(jnp.int32))
            return jax.lax.fori_loop(0, _NCH, body, jnp.int32(0))

        def bit_cond(st):
            bit, _, cnt = st
            return (bit < 32) & (cnt != k_eff)

        def bit_step(st):
            bit, upfx, cnt = st
            m = jax.lax.shift_left(jnp.int32(1), jnp.int32(31) - bit)
            ucand = upfx | m
            cand = ucand ^ _INT_MIN
            c2 = count_ge(cand)
            take = c2 >= k_eff
            return (bit + jnp.int32(1),
                    jnp.where(take, ucand, upfx),
                    jnp.where(take, c2, cnt))

        _, upfx, _ = jax.lax.while_loop(
            bit_cond, bit_step, (jnp.int32(0), jnp.int32(0), jnp.int32(_N)))
        thresh = upfx ^ _INT_MIN

        def tail(c, carry):
            cg, sg = carry
            ik = key_ref[pl.ds(c * _CH, _CH), :]
            gt = ik > thresh
            vv = _unkey(ik)
            return (cg + jnp.sum(gt.astype(jnp.int32)),
                    sg + jnp.sum(jnp.where(gt, vv, 0.0)))

        cnt_gt, sum_gt = jax.lax.fori_loop(
            0, _NCH, tail, (jnp.int32(0), jnp.float32(0.0)))

        t_val = _unkey(thresh)
        topk_sum = sum_gt + (k_eff - cnt_gt).astype(jnp.float32) * t_val
        denom = k_eff.astype(jnp.float32)

        lane = jax.lax.broadcasted_iota(jnp.int32, (1, 128), 1)
        row = jnp.where(lane == 0, pos_sum,
              jnp.where(lane == 1, acc_ref[1],
              jnp.where(lane == 2, topk_sum,
              jnp.where(lane == 3, denom, 0.0))))
        out_ref[...] = row


def _reg_kernel(rm_ref, tg_ref, pos_ref, out_ref, acc_ref):
    b = pl.program_id(0)
    h = pl.program_id(1)

    @pl.when((b == 0) & (h == 0))
    def _init():
        acc_ref[0] = 0.0
        out_ref[...] = jnp.zeros_like(out_ref)

    x = rm_ref[0]
    y = jnp.transpose(tg_ref[0], (2, 0, 1))
    pmt = pos_ref[0]
    pm70 = jnp.broadcast_to(pmt[:, None], (_A, 7, _HB, _W)).reshape(_C, _HB, _W)
    d = (x - y) * pm70
    ad = jnp.abs(d)
    f = jnp.where(ad < 1.0, 0.5 * d * d, ad - 0.5)
    acc_ref[0] += jnp.sum(f)

    @pl.when((b == _B - 1) & (h == _GH - 1))
    def _fin():
        out_ref[...] = jnp.full_like(out_ref, acc_ref[0])


def kernel(rm, psm, pos_equal_one, neg_equal_one, targets):
    row = pl.pallas_call(
        _cls_sel_kernel,
        grid=(_B, _GH2),
        in_specs=[
            pl.BlockSpec((1, _A, _BH2, _W), lambda b, h: (b, 0, h, 0)),
            pl.BlockSpec((1, _BH2, _W, _A), lambda b, h: (b, h, 0, 0)),
            pl.BlockSpec((1, _BH2, _W, _A), lambda b, h: (b, h, 0, 0)),
        ],
        out_specs=[
            pl.BlockSpec((1, 128), lambda b, h: (0, 0)),
            pl.BlockSpec((1, _A, _BH2, _W), lambda b, h: (b, 0, h, 0)),
        ],
        out_shape=[
            jax.ShapeDtypeStruct((1, 128), jnp.float32),
            jax.ShapeDtypeStruct((_B, _A, _H, _W), jnp.float32),
        ],
        scratch_shapes=[pltpu.VMEM((_KROWS, _W), jnp.int32),
                        pltpu.SMEM((2,), jnp.float32)],
    )(psm, pos_equal_one, neg_equal_one)
    row, pos_t = row

    reg = pl.pallas_call(
        _reg_kernel,
        grid=(_B, _GH),
        in_specs=[
            pl.BlockSpec((1, _C, _HB, _W), lambda b, h: (b, 0, h, 0)),
            pl.BlockSpec((1, _HB, _W, _C), lambda b, h: (b, h, 0, 0)),
            pl.BlockSpec((1, _A, _HB, _W), lambda b, h: (b, 0, h, 0)),
        ],
        out_specs=pl.BlockSpec((1, 1), lambda b, h: (0, 0)),
        out_shape=jax.ShapeDtypeStruct((1, 1), jnp.float32),
        scratch_shapes=[pltpu.SMEM((1,), jnp.float32)],
    )(rm, targets, pos_t)

    pos_sum = row[0, 0]
    clsp_sum = row[0, 1]
    topk_sum = row[0, 2]
    denom = row[0, 3]
    reg_sum = reg[0, 0]

    cls_pos_loss = _ALPHA * (clsp_sum / (pos_sum + 1e-6))
    cls_neg_loss = _BETA * (topk_sum / (denom + 1e-6))
    reg_loss = _GAMMA * (reg_sum / (pos_sum + 1e-6))
    conf_loss = cls_pos_loss + cls_neg_loss
    return (conf_loss, reg_loss, cls_pos_loss, cls_neg_loss)


# speedup vs baseline: 21.1687x; 1.0342x over previous
import jax
import jax.numpy as jnp
from jax.experimental import pallas as pl
from jax.experimental.pallas import tpu as pltpu

_NEG_RATIO = 1.0
_ALPHA = 1.5
_BETA = 1.0
_GAMMA = 2.0

_B, _H, _W, _A = 8, 200, 176, 10
_C = _A * 7
_N = _B * _H * _W * _A
_KROWS = _B * _A * _H
_BH2 = 40
_GH2 = _H // _BH2
_CH = 1600
_NCH = _KROWS // _CH
_HB = 40
_GH = _H // _HB

_INT_MIN = -2147483648
_POS_MASK = 0x7FFFFFFF


def _monokey(x):
    b = jax.lax.bitcast_convert_type(x, jnp.int32)
    return jnp.where(b < 0, b ^ _POS_MASK, b)


def _unkey(k):
    b = jnp.where(k < 0, k ^ _POS_MASK, k)
    return jax.lax.bitcast_convert_type(b, jnp.float32)


def _cls_sel_kernel(psm_ref, pos_ref, neg_ref, out_ref, post_ref, key_ref,
                    acc_ref):
    b = pl.program_id(0)
    h = pl.program_id(1)

    @pl.when((b == 0) & (h == 0))
    def _init():
        acc_ref[0] = 0.0
        acc_ref[1] = 0.0
        out_ref[...] = jnp.zeros_like(out_ref)

    x = psm_ref[0]
    post = jnp.transpose(pos_ref[0], (2, 0, 1))
    negt = jnp.transpose(neg_ref[0], (2, 0, 1))
    post_ref[0] = post
    p = jax.nn.sigmoid(x)
    acc_ref[0] += jnp.sum(post)
    acc_ref[1] += jnp.sum(-post * jnp.log(p + 1e-6))
    v = -negt * jnp.log(1.0 - p + 1e-6)
    keys = _monokey(v)
    base = b * (_A * _H) + h * _BH2
    for c in range(_A):
        key_ref[pl.ds(base + c * _H, _BH2), :] = keys[c]

    @pl.when((b == _B - 1) & (h == _GH2 - 1))
    def _finish():
        pos_sum = acc_ref[0]
        k_i = jnp.floor(_NEG_RATIO * (pos_sum + 1.0)).astype(jnp.int32)
        k_eff = jnp.minimum(k_i, _N)

        def count_ge(thresh):
            def body(c, acc):
                ik = key_ref[pl.ds(c * _CH, _CH), :]
                return acc + jnp.sum((ik >= thresh).astype(jnp.int32))
            return jax.lax.fori_loop(0, _NCH, body, jnp.int32(0))

        def bit_cond(st):
            bit, _, cnt = st
            return (bit < 32) & (cnt != k_eff)

        def bit_step(st):
            bit, upfx, cnt = st
            m = jax.lax.shift_left(jnp.int32(1), jnp.int32(31) - bit)
            ucand = upfx | m
            cand = ucand ^ _INT_MIN
            c2 = count_ge(cand)
            take = c2 >= k_eff
            return (bit + jnp.int32(1),
                    jnp.where(take, ucand, upfx),
                    jnp.where(take, c2, cnt))

        _, upfx, _ = jax.lax.while_loop(
            bit_cond, bit_step, (jnp.int32(0), jnp.int32(0), jnp.int32(_N)))
        thresh = upfx ^ _INT_MIN

        def tail(c, carry):
            cg, sg = carry
            ik = key_ref[pl.ds(c * _CH, _CH), :]
            gt = ik > thresh
            vv = _unkey(ik)
            return (cg + jnp.sum(gt.astype(jnp.int32)),
                    sg + jnp.sum(jnp.where(gt, vv, 0.0)))

        cnt_gt, sum_gt = jax.lax.fori_loop(
            0, _NCH, tail, (jnp.int32(0), jnp.float32(0.0)))

        t_val = _unkey(thresh)
        topk_sum = sum_gt + (k_eff - cnt_gt).astype(jnp.float32) * t_val
        denom = k_eff.astype(jnp.float32)

        lane = jax.lax.broadcasted_iota(jnp.int32, (1, 128), 1)
        row = jnp.where(lane == 0, pos_sum,
              jnp.where(lane == 1, acc_ref[1],
              jnp.where(lane == 2, topk_sum,
              jnp.where(lane == 3, denom, 0.0))))
        out_ref[...] = row


def _reg_kernel(rm_ref, tg_ref, pos_ref, out_ref, acc_ref):
    b = pl.program_id(0)
    h = pl.program_id(1)

    @pl.when((b == 0) & (h == 0))
    def _init():
        acc_ref[0] = 0.0
        out_ref[...] = jnp.zeros_like(out_ref)

    x = rm_ref[0]
    y = jnp.transpose(tg_ref[0], (2, 0, 1))
    pmt = pos_ref[0]
    pm70 = jnp.broadcast_to(pmt[:, None], (_A, 7, _HB, _W)).reshape(_C, _HB, _W)
    d = (x - y) * pm70
    ad = jnp.abs(d)
    f = jnp.where(ad < 1.0, 0.5 * d * d, ad - 0.5)
    acc_ref[0] += jnp.sum(f)

    @pl.when((b == _B - 1) & (h == _GH - 1))
    def _fin():
        out_ref[...] = jnp.full_like(out_ref, acc_ref[0])


def kernel(rm, psm, pos_equal_one, neg_equal_one, targets):
    row = pl.pallas_call(
        _cls_sel_kernel,
        grid=(_B, _GH2),
        in_specs=[
            pl.BlockSpec((1, _A, _BH2, _W), lambda b, h: (b, 0, h, 0)),
            pl.BlockSpec((1, _BH2, _W, _A), lambda b, h: (b, h, 0, 0)),
            pl.BlockSpec((1, _BH2, _W, _A), lambda b, h: (b, h, 0, 0)),
        ],
        out_specs=[
            pl.BlockSpec((1, 128), lambda b, h: (0, 0)),
            pl.BlockSpec((1, _A, _BH2, _W), lambda b, h: (b, 0, h, 0)),
        ],
        out_shape=[
            jax.ShapeDtypeStruct((1, 128), jnp.float32),
            jax.ShapeDtypeStruct((_B, _A, _H, _W), jnp.float32),
        ],
        scratch_shapes=[pltpu.VMEM((_KROWS, _W), jnp.int32),
                        pltpu.SMEM((2,), jnp.float32)],
    )(psm, pos_equal_one, neg_equal_one)
    row, pos_t = row

    reg = pl.pallas_call(
        _reg_kernel,
        grid=(_B, _GH),
        in_specs=[
            pl.BlockSpec((1, _C, _HB, _W), lambda b, h: (b, 0, h, 0)),
            pl.BlockSpec((1, _HB, _W, _C), lambda b, h: (b, h, 0, 0)),
            pl.BlockSpec((1, _A, _HB, _W), lambda b, h: (b, 0, h, 0)),
        ],
        out_specs=pl.BlockSpec((1, 1), lambda b, h: (0, 0)),
        out_shape=jax.ShapeDtypeStruct((1, 1), jnp.float32),
        scratch_shapes=[pltpu.SMEM((1,), jnp.float32)],
    )(rm, targets, pos_t)

    pos_sum = row[0, 0]
    clsp_sum = row[0, 1]
    topk_sum = row[0, 2]
    denom = row[0, 3]
    reg_sum = reg[0, 0]

    cls_pos_loss = _ALPHA * (clsp_sum / (pos_sum + 1e-6))
    cls_neg_loss = _BETA * (topk_sum / (denom + 1e-6))
    reg_loss = _GAMMA * (reg_sum / (pos_sum + 1e-6))
    conf_loss = cls_pos_loss + cls_neg_loss
    return (conf_loss, reg_loss, cls_pos_loss, cls_neg_loss)
